# Initial kernel scaffold; baseline (speedup 1.0000x reference)
import jax
import jax.numpy as jnp
from jax.experimental import pallas as pl


def kernel(protein_pos, protein_v, batch_protein, ligand_pos, ligand_v, batch_ligand, W_prot, b_prot, W_lig, b_lig, W_msg, b_msg, W_x, W_h1, b_h1, W_h2, b_h2, W_h3, b_h3):
    raise NotImplementedError("write your pallas kernel here")


# speedup vs baseline: 2.3778x; 2.3778x over previous
import functools

import jax
import jax.numpy as jnp
from jax.experimental import pallas as pl
from jax.experimental.pallas import tpu as pltpu

B = 128
NUM_CLASSES = 13
HIDDEN = 128
NP = 50000
NL = 10000
N = NP + NL
TILE = 2000
NT = N // TILE
NTP = NP // TILE
NTL = NL // TILE
NSTEPS = 4 * NT + 1
_LN2 = 0.6931471805599453


def _sigmoid(x):
    return 1.0 / (1.0 + jnp.exp(-x))


def _ssp(x):
    return jnp.log1p(jnp.exp(-jnp.abs(x))) + jnp.maximum(x, 0.0) - _LN2


def _dot(a, b):
    return jax.lax.dot_general(
        a, b, (((1,), (0,)), ((), ())),
        preferred_element_type=jnp.float32,
        precision=jax.lax.Precision.HIGHEST)


def _body(pos_ref, pv_ref, lv_ref, br_ref, bc_ref,
          wp_ref, bp_ref, wl_ref, bl_ref,
          a_ref, bmat_ref, bmsg_ref, wx_ref,
          wh1p_ref, wh1h_ref, bh1_ref, wh2_ref, bh2_ref, wh3_ref, bh3_ref,
          out_ref,
          h_ref, acc_ref, bias_ref, statp_ref, statl_ref, pool_ref):
    i = pl.program_id(0)
    tile = i % NT
    phase = i // NT
    row_off = pl.multiple_of(tile * TILE, TILE)
    is_prot = tile < NTP

    @pl.when(i == 0)
    def _init():
        acc_ref[...] = jnp.zeros_like(acc_ref)
        statp_ref[...] = jnp.zeros_like(statp_ref)
        statl_ref[...] = jnp.zeros_like(statl_ref)
        pool_ref[...] = jnp.zeros_like(pool_ref)

    def _seg_mean():
        cnt = statp_ref[:, 3:4] + statl_ref[:, 3:4]
        return acc_ref[...] / jnp.maximum(cnt, 1.0)

    @pl.when(i == NT)
    def _b0():
        m = _seg_mean()
        bias_ref[...] = _dot(m, bmat_ref[0]) + bmsg_ref[0]
        acc_ref[...] = jnp.zeros_like(acc_ref)

    @pl.when(i == 2 * NT)
    def _b1():
        m = _seg_mean()
        pool_ref[...] += _dot(m, wx_ref[0])
        bias_ref[...] = _dot(m, bmat_ref[1]) + bmsg_ref[1]
        acc_ref[...] = jnp.zeros_like(acc_ref)

    @pl.when(i == 3 * NT)
    def _b2():
        m = _seg_mean()
        pool_ref[...] += _dot(m, wx_ref[1])
        bias_ref[...] = _dot(m, bmat_ref[2]) + bmsg_ref[2]
        acc_ref[...] = jnp.zeros_like(acc_ref)

    ids_r = br_ref[0]
    ids_c = bc_ref[0]
    oh_t = (jax.lax.broadcasted_iota(jnp.int32, (B, TILE), 0) == ids_r
            ).astype(jnp.float32)
    oh = (jax.lax.broadcasted_iota(jnp.int32, (TILE, B), 1) == ids_c
          ).astype(jnp.float32)

    @pl.when(phase == 0)
    def _embed():
        h0_p = _dot(pv_ref[...], wp_ref[...]) + bp_ref[...]
        ohc = (jax.lax.broadcasted_iota(jnp.int32, (TILE, NUM_CLASSES), 1)
               == lv_ref[0]).astype(jnp.float32)
        h0_l = _dot(ohc, wl_ref[...]) + bl_ref[...]
        h0 = jnp.where(is_prot, h0_p, h0_l)
        h_ref[pl.ds(row_off, TILE), :] = h0
        acc_ref[...] += _dot(oh_t, h0)
        po = jnp.concatenate(
            [pos_ref[...], jnp.ones((TILE, 1), jnp.float32)], axis=1)
        st = _dot(oh_t, po)
        statp_ref[...] += jnp.where(is_prot, st, 0.0)
        statl_ref[...] += jnp.where(is_prot, 0.0, st)

    def _layer(l):
        h_old = h_ref[pl.ds(row_off, TILE), :]
        pre = _dot(h_old, a_ref[l]) + _dot(oh, bias_ref[...])
        h_new = h_old + pre * _sigmoid(pre)
        h_ref[pl.ds(row_off, TILE), :] = h_new
        acc_ref[...] += _dot(oh_t, h_new)

    @pl.when(phase == 1)
    def _l0():
        _layer(0)

    @pl.when(phase == 2)
    def _l1():
        _layer(1)

    @pl.when(phase == 3)
    def _l2():
        _layer(2)

    @pl.when(i == 4 * NT)
    def _head():
        m3 = _seg_mean()
        pool = pool_ref[...] + _dot(m3, wx_ref[2])
        sum_p = statp_ref[:, 0:3]
        sum_l = statl_ref[:, 0:3]
        cnt_p = statp_ref[:, 3:4]
        cnt_all = cnt_p + statl_ref[:, 3:4]
        offset = sum_p / jnp.maximum(cnt_p, 1.0)
        pos_mean = (sum_p + sum_l - cnt_all * offset) / jnp.maximum(
            cnt_all, 1.0)
        pos_final = pos_mean + pool[:, 0:3]
        z1 = _dot(pos_final, wh1p_ref[...]) + _dot(m3, wh1h_ref[...]) \
            + bh1_ref[...]
        a1 = _ssp(z1)
        z2 = _dot(a1, wh2_ref[...]) + bh2_ref[...]
        a2 = _ssp(z2)
        out_ref[...] = _dot(a2, wh3_ref[...]) + bh3_ref[...]


@jax.jit
def kernel(protein_pos, protein_v, batch_protein, ligand_pos, ligand_v,
           batch_ligand, W_prot, b_prot, W_lig, b_lig, W_msg, b_msg, W_x,
           W_h1, b_h1, W_h2, b_h2, W_h3, b_h3):
    f32 = jnp.float32
    pos_all = jnp.concatenate([protein_pos, ligand_pos], axis=0)
    batch_all = jnp.concatenate([batch_protein, batch_ligand], axis=0)
    batch_r = batch_all.astype(jnp.int32).reshape(NT, 1, TILE)
    batch_c = batch_all.astype(jnp.int32).reshape(NT, TILE, 1)
    lv_c = ligand_v.astype(jnp.int32).reshape(NTL, TILE, 1)
    wp = jnp.pad(W_prot, ((0, 0), (0, 1)))
    bp = jnp.concatenate([b_prot, jnp.zeros((1,), f32)]).reshape(1, HIDDEN)
    wl = jnp.pad(W_lig, ((0, 0), (0, 1)))
    bl = jnp.concatenate([b_lig, jnp.ones((1,), f32)]).reshape(1, HIDDEN)
    a_w = W_msg[:, :HIDDEN, :]
    b_w = W_msg[:, HIDDEN:, :]
    bmsg = b_msg.reshape(3, 1, HIDDEN)
    wx = jnp.pad(W_x, ((0, 0), (0, 0), (0, 1)))
    wh1p = W_h1[0:3]
    wh1h = W_h1[3:]
    bh1 = b_h1.reshape(1, -1)
    bh2 = b_h2.reshape(1, -1)
    bh3 = b_h3.reshape(1, 1)

    def im_const(i):
        return (0, 0)

    def im_pos(i):
        return (jnp.where(i // NT == 0, jnp.minimum(i % NT, NT - 1), NT - 1),
                0)

    def im_pv(i):
        return (jnp.where(i // NT == 0, jnp.minimum(i % NT, NTP - 1),
                          NTP - 1), 0)

    def im_lv(i):
        return (jnp.where(i // NT == 0,
                          jnp.clip(i % NT - NTP, 0, NTL - 1), NTL - 1), 0, 0)

    def im_b3(i):
        return (jnp.minimum(i % NT, NT - 1), 0, 0)

    in_specs = [
            pl.BlockSpec((TILE, 3), im_pos),
            pl.BlockSpec((TILE, 27), im_pv),
            pl.BlockSpec((1, TILE, 1), im_lv),
            pl.BlockSpec((1, 1, TILE), im_b3),
            pl.BlockSpec((1, TILE, 1), im_b3),
            pl.BlockSpec((27, HIDDEN), im_const),
            pl.BlockSpec((1, HIDDEN), im_const),
            pl.BlockSpec((NUM_CLASSES, HIDDEN), im_const),
            pl.BlockSpec((1, HIDDEN), im_const),
            pl.BlockSpec((3, HIDDEN, HIDDEN), lambda i: (0, 0, 0)),
            pl.BlockSpec((3, HIDDEN, HIDDEN), lambda i: (0, 0, 0)),
            pl.BlockSpec((3, 1, HIDDEN), lambda i: (0, 0, 0)),
            pl.BlockSpec((3, HIDDEN, 4), lambda i: (0, 0, 0)),
            pl.BlockSpec((3, 2 * HIDDEN), im_const),
            pl.BlockSpec((HIDDEN, 2 * HIDDEN), im_const),
            pl.BlockSpec((1, 2 * HIDDEN), im_const),
            pl.BlockSpec((2 * HIDDEN, HIDDEN), im_const),
            pl.BlockSpec((1, HIDDEN), im_const),
            pl.BlockSpec((HIDDEN, 1), im_const),
            pl.BlockSpec((1, 1), im_const),
        ]

    return pl.pallas_call(
        _body,
        grid=(NSTEPS,),
        in_specs=in_specs,
        out_specs=pl.BlockSpec((B, 1), im_const),
        out_shape=jax.ShapeDtypeStruct((B, 1), f32),
        scratch_shapes=[
            pltpu.VMEM((N, HIDDEN), f32),
            pltpu.VMEM((B, HIDDEN), f32),
            pltpu.VMEM((B, HIDDEN), f32),
            pltpu.VMEM((B, 4), f32),
            pltpu.VMEM((B, 4), f32),
            pltpu.VMEM((B, 4), f32),
        ],
    )(pos_all, protein_v, lv_c, batch_r, batch_c, wp, bp, wl, bl,
      a_w, b_w, bmsg, wx, wh1p, wh1h, bh1, W_h2, bh2, W_h3, bh3)


# speedup vs baseline: 3.5414x; 1.4894x over previous
import functools

import jax
import jax.numpy as jnp
from jax.experimental import pallas as pl
from jax.experimental.pallas import tpu as pltpu

B = 128
NUM_CLASSES = 13
HIDDEN = 128
NP = 50000
NL = 10000
N = NP + NL
TILE = 2000
NT = N // TILE
NTP = NP // TILE
NTL = NL // TILE
NSTEPS = 4 * NT + 1
_LN2 = 0.6931471805599453


def _sigmoid(x):
    return 1.0 / (1.0 + jnp.exp(-x))


def _ssp(x):
    return jnp.log1p(jnp.exp(-jnp.abs(x))) + jnp.maximum(x, 0.0) - _LN2


def _dot(a, b):
    return jax.lax.dot_general(
        a, b, (((1,), (0,)), ((), ())),
        preferred_element_type=jnp.float32,
        precision=jax.lax.Precision.HIGHEST)


def _dot1(a, b):
    return jax.lax.dot_general(
        a, b, (((1,), (0,)), ((), ())),
        preferred_element_type=jnp.float32,
        precision=jax.lax.Precision.DEFAULT)


def _split(x):
    hi = x.astype(jnp.bfloat16)
    lo = (x - hi.astype(jnp.float32)).astype(jnp.bfloat16)
    return hi, lo


def _dot3(a, b_hi, b_lo):
    a_hi, a_lo = _split(a)
    return (_dot1(a_hi, b_hi) + _dot1(a_hi, b_lo)) + _dot1(a_lo, b_hi)


def _dot2(a_exact, b_hi, b_lo):
    return _dot1(a_exact, b_hi) + _dot1(a_exact, b_lo)


def _dot2s(a_exact, b):
    b_hi, b_lo = _split(b)
    return _dot1(a_exact, b_hi) + _dot1(a_exact, b_lo)


def _body(pos_ref, pv_ref, lv_ref, br_ref, bc_ref,
          wph_ref, wpl_ref, bp_ref, wlh_ref, wll_ref, bl_ref,
          ah_ref, al_ref, bmat_ref, bmsg_ref, wx_ref,
          wh1p_ref, wh1h_ref, bh1_ref, wh2_ref, bh2_ref, wh3_ref, bh3_ref,
          out_ref,
          h_ref, acc_ref, bias_h_ref, bias_l_ref,
          statp_ref, statl_ref, pool_ref):
    i = pl.program_id(0)
    tile = i % NT
    phase = i // NT
    row_off = pl.multiple_of(tile * TILE, TILE)
    is_prot = tile < NTP

    @pl.when(i == 0)
    def _init():
        acc_ref[...] = jnp.zeros_like(acc_ref)
        statp_ref[...] = jnp.zeros_like(statp_ref)
        statl_ref[...] = jnp.zeros_like(statl_ref)
        pool_ref[...] = jnp.zeros_like(pool_ref)

    def _seg_mean():
        cnt = statp_ref[:, 3:4] + statl_ref[:, 3:4]
        return acc_ref[...] / jnp.maximum(cnt, 1.0)

    def _freeze_bias(m, l):
        bias = _dot(m, bmat_ref[l]) + bmsg_ref[l]
        hi, lo = _split(bias)
        bias_h_ref[...] = hi
        bias_l_ref[...] = lo
        acc_ref[...] = jnp.zeros_like(acc_ref)

    @pl.when(i == NT)
    def _b0():
        _freeze_bias(_seg_mean(), 0)

    @pl.when(i == 2 * NT)
    def _b1():
        m = _seg_mean()
        pool_ref[...] += _dot(m, wx_ref[0])
        _freeze_bias(m, 1)

    @pl.when(i == 3 * NT)
    def _b2():
        m = _seg_mean()
        pool_ref[...] += _dot(m, wx_ref[1])
        _freeze_bias(m, 2)

    ids_r = br_ref[0]
    ids_c = bc_ref[0]
    oh_t = (jax.lax.broadcasted_iota(jnp.int32, (B, TILE), 0) == ids_r
            ).astype(jnp.bfloat16)
    oh = (jax.lax.broadcasted_iota(jnp.int32, (TILE, B), 1) == ids_c
          ).astype(jnp.bfloat16)

    @pl.when(phase == 0)
    def _embed():
        h0_p = _dot3(pv_ref[...], wph_ref[...], wpl_ref[...]) + bp_ref[...]
        ohc = (jax.lax.broadcasted_iota(jnp.int32, (TILE, NUM_CLASSES), 1)
               == lv_ref[0]).astype(jnp.bfloat16)
        h0_l = _dot2(ohc, wlh_ref[...], wll_ref[...]) + bl_ref[...]
        h0 = jnp.where(is_prot, h0_p, h0_l)
        h_ref[pl.ds(row_off, TILE), :] = h0
        acc_ref[...] += _dot2s(oh_t, h0)
        po = jnp.concatenate(
            [pos_ref[...], jnp.ones((TILE, 1), jnp.float32)], axis=1)
        st = _dot2s(oh_t, po)
        statp_ref[...] += jnp.where(is_prot, st, 0.0)
        statl_ref[...] += jnp.where(is_prot, 0.0, st)

    def _layer(l):
        h_old = h_ref[pl.ds(row_off, TILE), :]
        pre = _dot3(h_old, ah_ref[l], al_ref[l]) \
            + _dot2(oh, bias_h_ref[...], bias_l_ref[...])
        h_new = h_old + pre * _sigmoid(pre)
        h_ref[pl.ds(row_off, TILE), :] = h_new
        acc_ref[...] += _dot2s(oh_t, h_new)

    @pl.when(phase == 1)
    def _l0():
        _layer(0)

    @pl.when(phase == 2)
    def _l1():
        _layer(1)

    @pl.when(phase == 3)
    def _l2():
        _layer(2)

    @pl.when(i == 4 * NT)
    def _head():
        m3 = _seg_mean()
        pool = pool_ref[...] + _dot(m3, wx_ref[2])
        sum_p = statp_ref[:, 0:3]
        sum_l = statl_ref[:, 0:3]
        cnt_p = statp_ref[:, 3:4]
        cnt_all = cnt_p + statl_ref[:, 3:4]
        offset = sum_p / jnp.maximum(cnt_p, 1.0)
        pos_mean = (sum_p + sum_l - cnt_all * offset) / jnp.maximum(
            cnt_all, 1.0)
        pos_final = pos_mean + pool[:, 0:3]
        z1 = _dot(pos_final, wh1p_ref[...]) + _dot(m3, wh1h_ref[...]) \
            + bh1_ref[...]
        a1 = _ssp(z1)
        z2 = _dot(a1, wh2_ref[...]) + bh2_ref[...]
        a2 = _ssp(z2)
        out_ref[...] = _dot(a2, wh3_ref[...]) + bh3_ref[...]


@jax.jit
def kernel(protein_pos, protein_v, batch_protein, ligand_pos, ligand_v,
           batch_ligand, W_prot, b_prot, W_lig, b_lig, W_msg, b_msg, W_x,
           W_h1, b_h1, W_h2, b_h2, W_h3, b_h3):
    f32 = jnp.float32
    pos_all = jnp.concatenate([protein_pos, ligand_pos], axis=0)
    batch_all = jnp.concatenate([batch_protein, batch_ligand], axis=0)
    batch_r = batch_all.astype(jnp.int32).reshape(NT, 1, TILE)
    batch_c = batch_all.astype(jnp.int32).reshape(NT, TILE, 1)
    lv_c = ligand_v.astype(jnp.int32).reshape(NTL, TILE, 1)
    def split(x):
        hi = x.astype(jnp.bfloat16)
        return hi, (x - hi.astype(f32)).astype(jnp.bfloat16)

    wp_hi, wp_lo = split(jnp.pad(W_prot, ((0, 0), (0, 1))))
    bp = jnp.concatenate([b_prot, jnp.zeros((1,), f32)]).reshape(1, HIDDEN)
    wl_hi, wl_lo = split(jnp.pad(W_lig, ((0, 0), (0, 1))))
    bl = jnp.concatenate([b_lig, jnp.ones((1,), f32)]).reshape(1, HIDDEN)
    a_hi, a_lo = split(W_msg[:, :HIDDEN, :])
    b_w = W_msg[:, HIDDEN:, :]
    bmsg = b_msg.reshape(3, 1, HIDDEN)
    wx = jnp.pad(W_x, ((0, 0), (0, 0), (0, 1)))
    wh1p = W_h1[0:3]
    wh1h = W_h1[3:]
    bh1 = b_h1.reshape(1, -1)
    bh2 = b_h2.reshape(1, -1)
    bh3 = b_h3.reshape(1, 1)

    def im_const(i):
        return (0, 0)

    def im_pos(i):
        return (jnp.where(i // NT == 0, jnp.minimum(i % NT, NT - 1), NT - 1),
                0)

    def im_pv(i):
        return (jnp.where(i // NT == 0, jnp.minimum(i % NT, NTP - 1),
                          NTP - 1), 0)

    def im_lv(i):
        return (jnp.where(i // NT == 0,
                          jnp.clip(i % NT - NTP, 0, NTL - 1), NTL - 1), 0, 0)

    def im_b3(i):
        return (jnp.minimum(i % NT, NT - 1), 0, 0)

    in_specs = [
            pl.BlockSpec((TILE, 3), im_pos),
            pl.BlockSpec((TILE, 27), im_pv),
            pl.BlockSpec((1, TILE, 1), im_lv),
            pl.BlockSpec((1, 1, TILE), im_b3),
            pl.BlockSpec((1, TILE, 1), im_b3),
            pl.BlockSpec((27, HIDDEN), im_const),
            pl.BlockSpec((27, HIDDEN), im_const),
            pl.BlockSpec((1, HIDDEN), im_const),
            pl.BlockSpec((NUM_CLASSES, HIDDEN), im_const),
            pl.BlockSpec((NUM_CLASSES, HIDDEN), im_const),
            pl.BlockSpec((1, HIDDEN), im_const),
            pl.BlockSpec((3, HIDDEN, HIDDEN), lambda i: (0, 0, 0)),
            pl.BlockSpec((3, HIDDEN, HIDDEN), lambda i: (0, 0, 0)),
            pl.BlockSpec((3, HIDDEN, HIDDEN), lambda i: (0, 0, 0)),
            pl.BlockSpec((3, 1, HIDDEN), lambda i: (0, 0, 0)),
            pl.BlockSpec((3, HIDDEN, 4), lambda i: (0, 0, 0)),
            pl.BlockSpec((3, 2 * HIDDEN), im_const),
            pl.BlockSpec((HIDDEN, 2 * HIDDEN), im_const),
            pl.BlockSpec((1, 2 * HIDDEN), im_const),
            pl.BlockSpec((2 * HIDDEN, HIDDEN), im_const),
            pl.BlockSpec((1, HIDDEN), im_const),
            pl.BlockSpec((HIDDEN, 1), im_const),
            pl.BlockSpec((1, 1), im_const),
        ]

    return pl.pallas_call(
        _body,
        grid=(NSTEPS,),
        in_specs=in_specs,
        out_specs=pl.BlockSpec((B, 1), im_const),
        out_shape=jax.ShapeDtypeStruct((B, 1), f32),
        scratch_shapes=[
            pltpu.VMEM((N, HIDDEN), f32),
            pltpu.VMEM((B, HIDDEN), f32),
            pltpu.VMEM((B, HIDDEN), jnp.bfloat16),
            pltpu.VMEM((B, HIDDEN), jnp.bfloat16),
            pltpu.VMEM((B, 4), f32),
            pltpu.VMEM((B, 4), f32),
            pltpu.VMEM((B, 4), f32),
        ],
    )(pos_all, protein_v, lv_c, batch_r, batch_c, wp_hi, wp_lo, bp,
      wl_hi, wl_lo, bl, a_hi, a_lo, b_w, bmsg, wx,
      wh1p, wh1h, bh1, W_h2, bh2, W_h3, bh3)


# speedup vs baseline: 4.8432x; 1.3676x over previous
import functools

import jax
import jax.numpy as jnp
from jax.experimental import pallas as pl
from jax.experimental.pallas import tpu as pltpu

B = 128
NUM_CLASSES = 13
HIDDEN = 128
NP = 50000
NL = 10000
N = NP + NL
TILE = 2000
NT = N // TILE
NTP = NP // TILE
NTL = NL // TILE
NSTEPS = 4 * NT + 1
_LN2 = 0.6931471805599453


def _sigmoid(x):
    return 1.0 / (1.0 + jnp.exp(-x))


def _ssp(x):
    return jnp.log1p(jnp.exp(-jnp.abs(x))) + jnp.maximum(x, 0.0) - _LN2


def _dot(a, b):
    return jax.lax.dot_general(
        a, b, (((1,), (0,)), ((), ())),
        preferred_element_type=jnp.float32,
        precision=jax.lax.Precision.HIGHEST)


def _dot1(a, b):
    return jax.lax.dot_general(
        a, b, (((1,), (0,)), ((), ())),
        preferred_element_type=jnp.float32,
        precision=jax.lax.Precision.DEFAULT)


def _split(x):
    hi = x.astype(jnp.bfloat16)
    lo = (x - hi.astype(jnp.float32)).astype(jnp.bfloat16)
    return hi, lo


def _dot2r(a, b_exact):
    a_hi, a_lo = _split(a)
    return _dot1(a_hi, b_exact) + _dot1(a_lo, b_exact)


def _dot2(a_exact, b_hi, b_lo):
    return _dot1(a_exact, b_hi) + _dot1(a_exact, b_lo)


def _dot2s(a_exact, b):
    b_hi, b_lo = _split(b)
    return _dot1(a_exact, b_hi) + _dot1(a_exact, b_lo)


def _dotT1(a, b):
    return jax.lax.dot_general(
        a, b, (((0,), (0,)), ((), ())),
        preferred_element_type=jnp.float32,
        precision=jax.lax.Precision.DEFAULT)


def _dotT2s(a_exact, b):
    b_hi, b_lo = _split(b)
    return _dotT1(a_exact, b_hi) + _dotT1(a_exact, b_lo)


def _body(pos_ref, pv_ref, lv_ref, bc_ref,
          wp_ref, bp_ref, wl_ref, bl_ref,
          a_ref, bmat_ref, bmsg_ref, wx_ref,
          wh1p_ref, wh1h_ref, bh1_ref, wh2_ref, bh2_ref, wh3_ref, bh3_ref,
          out_ref,
          h_ref, oh_ref, acc_ref, bias_h_ref, bias_l_ref,
          statp_ref, statl_ref, pool_ref):
    i = pl.program_id(0)
    tile = i % NT
    phase = i // NT
    row_off = pl.multiple_of(tile * TILE, TILE)
    is_prot = tile < NTP

    @pl.when(i == 0)
    def _init():
        acc_ref[...] = jnp.zeros_like(acc_ref)
        statp_ref[...] = jnp.zeros_like(statp_ref)
        statl_ref[...] = jnp.zeros_like(statl_ref)
        pool_ref[...] = jnp.zeros_like(pool_ref)

    def _seg_mean():
        cnt = statp_ref[:, 3:4] + statl_ref[:, 3:4]
        return acc_ref[...] / jnp.maximum(cnt, 1.0)

    def _freeze_bias(m, l):
        bias = _dot1(m.astype(jnp.bfloat16), bmat_ref[l]) + bmsg_ref[l]
        hi, lo = _split(bias)
        bias_h_ref[...] = hi
        bias_l_ref[...] = lo
        acc_ref[...] = jnp.zeros_like(acc_ref)

    @pl.when(i == NT)
    def _b0():
        _freeze_bias(_seg_mean(), 0)

    @pl.when(i == 2 * NT)
    def _b1():
        m = _seg_mean()
        pool_ref[...] += _dot2r(m, wx_ref[0])
        _freeze_bias(m, 1)

    @pl.when(i == 3 * NT)
    def _b2():
        m = _seg_mean()
        pool_ref[...] += _dot2r(m, wx_ref[1])
        _freeze_bias(m, 2)

    @pl.when(phase == 0)
    def _embed():
        ids_c = bc_ref[0]
        oh = (jax.lax.broadcasted_iota(jnp.int32, (TILE, B), 1) == ids_c
              ).astype(jnp.bfloat16)
        oh_ref[pl.ds(row_off, TILE), :] = oh
        h0_p = _dot1(pv_ref[...].astype(jnp.bfloat16), wp_ref[...]) \
            + bp_ref[...]
        ohc = (jax.lax.broadcasted_iota(jnp.int32, (TILE, NUM_CLASSES), 1)
               == lv_ref[0]).astype(jnp.bfloat16)
        h0_l = _dot1(ohc, wl_ref[...]) + bl_ref[...]
        h0 = jnp.where(is_prot, h0_p, h0_l)
        h_ref[pl.ds(row_off, TILE), :] = h0
        acc_ref[...] += _dotT2s(oh, h0)
        po = jnp.concatenate(
            [pos_ref[...], jnp.ones((TILE, 1), jnp.float32)], axis=1)
        st = _dotT2s(oh, po)
        statp_ref[...] += jnp.where(is_prot, st, 0.0)
        statl_ref[...] += jnp.where(is_prot, 0.0, st)

    def _layer(l):
        h_old = h_ref[pl.ds(row_off, TILE), :]
        oh = oh_ref[pl.ds(row_off, TILE), :]
        pre = _dot1(h_old.astype(jnp.bfloat16), a_ref[l]) \
            + _dot2(oh, bias_h_ref[...], bias_l_ref[...])
        h_new = h_old + pre * _sigmoid(pre)
        if l < 2:
            h_ref[pl.ds(row_off, TILE), :] = h_new
        acc_ref[...] += _dotT2s(oh, h_new)

    @pl.when(phase == 1)
    def _l0():
        _layer(0)

    @pl.when(phase == 2)
    def _l1():
        _layer(1)

    @pl.when(phase == 3)
    def _l2():
        _layer(2)

    @pl.when(i == 4 * NT)
    def _head():
        m3 = _seg_mean()
        pool = pool_ref[...] + _dot2r(m3, wx_ref[2])
        sum_p = statp_ref[:, 0:3]
        sum_l = statl_ref[:, 0:3]
        cnt_p = statp_ref[:, 3:4]
        cnt_all = cnt_p + statl_ref[:, 3:4]
        offset = sum_p / jnp.maximum(cnt_p, 1.0)
        pos_mean = (sum_p + sum_l - cnt_all * offset) / jnp.maximum(
            cnt_all, 1.0)
        pos_final = pos_mean + pool[:, 0:3]
        bf = jnp.bfloat16
        z1 = _dot1(pos_final.astype(bf), wh1p_ref[...]) \
            + _dot1(m3.astype(bf), wh1h_ref[...]) + bh1_ref[...]
        a1 = _ssp(z1)
        z2 = _dot1(a1.astype(bf), wh2_ref[...]) + bh2_ref[...]
        a2 = _ssp(z2)
        out_ref[...] = _dot1(a2.astype(bf), wh3_ref[...]) + bh3_ref[...]


@jax.jit
def kernel(protein_pos, protein_v, batch_protein, ligand_pos, ligand_v,
           batch_ligand, W_prot, b_prot, W_lig, b_lig, W_msg, b_msg, W_x,
           W_h1, b_h1, W_h2, b_h2, W_h3, b_h3):
    f32 = jnp.float32
    pos_all = jnp.concatenate([protein_pos, ligand_pos], axis=0)
    batch_all = jnp.concatenate([batch_protein, batch_ligand], axis=0)
    batch_c = batch_all.astype(jnp.int32).reshape(NT, TILE, 1)
    lv_c = ligand_v.astype(jnp.int32).reshape(NTL, TILE, 1)
    bf = jnp.bfloat16
    wp = jnp.pad(W_prot, ((0, 0), (0, 1))).astype(bf)
    bp = jnp.concatenate([b_prot, jnp.zeros((1,), f32)]).reshape(1, HIDDEN)
    wl = jnp.pad(W_lig, ((0, 0), (0, 1))).astype(bf)
    bl = jnp.concatenate([b_lig, jnp.ones((1,), f32)]).reshape(1, HIDDEN)
    a_w = W_msg[:, :HIDDEN, :].astype(bf)
    b_w = W_msg[:, HIDDEN:, :].astype(bf)
    bmsg = b_msg.reshape(3, 1, HIDDEN)
    wx = jnp.pad(W_x, ((0, 0), (0, 0), (0, 1))).astype(bf)
    wh1p = W_h1[0:3].astype(bf)
    wh1h = W_h1[3:].astype(bf)
    bh1 = b_h1.reshape(1, -1)
    bh2 = b_h2.reshape(1, -1)
    bh3 = b_h3.reshape(1, 1)
    wh2 = W_h2.astype(bf)
    wh3 = W_h3.astype(bf)

    def im_const(i):
        return (0, 0)

    def im_pos(i):
        return (jnp.where(i // NT == 0, jnp.minimum(i % NT, NT - 1), NT - 1),
                0)

    def im_pv(i):
        return (jnp.where(i // NT == 0, jnp.minimum(i % NT, NTP - 1),
                          NTP - 1), 0)

    def im_lv(i):
        return (jnp.where(i // NT == 0,
                          jnp.clip(i % NT - NTP, 0, NTL - 1), NTL - 1), 0, 0)

    def im_b3(i):
        return (jnp.where(i // NT == 0, jnp.minimum(i % NT, NT - 1),
                          NT - 1), 0, 0)

    in_specs = [
            pl.BlockSpec((TILE, 3), im_pos),
            pl.BlockSpec((TILE, 27), im_pv),
            pl.BlockSpec((1, TILE, 1), im_lv),
            pl.BlockSpec((1, TILE, 1), im_b3),
            pl.BlockSpec((27, HIDDEN), im_const),
            pl.BlockSpec((1, HIDDEN), im_const),
            pl.BlockSpec((NUM_CLASSES, HIDDEN), im_const),
            pl.BlockSpec((1, HIDDEN), im_const),
            pl.BlockSpec((3, HIDDEN, HIDDEN), lambda i: (0, 0, 0)),
            pl.BlockSpec((3, HIDDEN, HIDDEN), lambda i: (0, 0, 0)),
            pl.BlockSpec((3, 1, HIDDEN), lambda i: (0, 0, 0)),
            pl.BlockSpec((3, HIDDEN, 4), lambda i: (0, 0, 0)),
            pl.BlockSpec((3, 2 * HIDDEN), im_const),
            pl.BlockSpec((HIDDEN, 2 * HIDDEN), im_const),
            pl.BlockSpec((1, 2 * HIDDEN), im_const),
            pl.BlockSpec((2 * HIDDEN, HIDDEN), im_const),
            pl.BlockSpec((1, HIDDEN), im_const),
            pl.BlockSpec((HIDDEN, 1), im_const),
            pl.BlockSpec((1, 1), im_const),
        ]

    return pl.pallas_call(
        _body,
        grid=(NSTEPS,),
        in_specs=in_specs,
        out_specs=pl.BlockSpec((B, 1), im_const),
        out_shape=jax.ShapeDtypeStruct((B, 1), f32),
        scratch_shapes=[
            pltpu.VMEM((N, HIDDEN), f32),
            pltpu.VMEM((N, HIDDEN), jnp.bfloat16),
            pltpu.VMEM((B, HIDDEN), f32),
            pltpu.VMEM((B, HIDDEN), jnp.bfloat16),
            pltpu.VMEM((B, HIDDEN), jnp.bfloat16),
            pltpu.VMEM((B, 4), f32),
            pltpu.VMEM((B, 4), f32),
            pltpu.VMEM((B, 4), f32),
        ],
    )(pos_all, protein_v, lv_c, batch_c, wp, bp, wl, bl,
      a_w, b_w, bmsg, wx, wh1p, wh1h, bh1, wh2, bh2, wh3, bh3)


# speedup vs baseline: 5.5407x; 1.1440x over previous
import functools

import jax
import jax.numpy as jnp
from jax.experimental import pallas as pl
from jax.experimental.pallas import tpu as pltpu

B = 128
NUM_CLASSES = 13
HIDDEN = 128
NP = 50000
NL = 10000
N = NP + NL
TILE = 2000
NT = N // TILE
NTP = NP // TILE
NTL = NL // TILE
NSTEPS = 4 * NT + 1
_LN2 = 0.6931471805599453


def _sigmoid(x):
    return 1.0 / (1.0 + jnp.exp(-x))


def _ssp(x):
    return jnp.log1p(jnp.exp(-jnp.abs(x))) + jnp.maximum(x, 0.0) - _LN2


def _dot(a, b):
    return jax.lax.dot_general(
        a, b, (((1,), (0,)), ((), ())),
        preferred_element_type=jnp.float32,
        precision=jax.lax.Precision.HIGHEST)


def _dot1(a, b):
    return jax.lax.dot_general(
        a, b, (((1,), (0,)), ((), ())),
        preferred_element_type=jnp.float32,
        precision=jax.lax.Precision.DEFAULT)


def _split(x):
    hi = x.astype(jnp.bfloat16)
    lo = (x - hi.astype(jnp.float32)).astype(jnp.bfloat16)
    return hi, lo


def _dot2r(a, b_exact):
    a_hi, a_lo = _split(a)
    return _dot1(a_hi, b_exact) + _dot1(a_lo, b_exact)


def _dot2(a_exact, b_hi, b_lo):
    return _dot1(a_exact, b_hi) + _dot1(a_exact, b_lo)


def _dot2s(a_exact, b):
    b_hi, b_lo = _split(b)
    return _dot1(a_exact, b_hi) + _dot1(a_exact, b_lo)


def _dotT1(a, b):
    return jax.lax.dot_general(
        a, b, (((0,), (0,)), ((), ())),
        preferred_element_type=jnp.float32,
        precision=jax.lax.Precision.DEFAULT)


def _dotT2s(a_exact, b):
    b_hi, b_lo = _split(b)
    return _dotT1(a_exact, b_hi) + _dotT1(a_exact, b_lo)


def _body(pos_ref, pv_ref, lv_ref, bc_ref,
          wp_ref, bp_ref, wl_ref, bl_ref,
          a_ref, bmat_ref, bmsg_ref, wx_ref,
          wh1p_ref, wh1h_ref, bh1_ref, wh2_ref, bh2_ref, wh3_ref, bh3_ref,
          out_ref,
          h_ref, oh_ref, acc_ref, bias_h_ref, bias_l_ref,
          statp_ref, statl_ref, pool_ref):
    i = pl.program_id(0)
    tile = i % NT
    phase = i // NT
    row_off = pl.multiple_of(tile * TILE, TILE)
    is_prot = tile < NTP

    @pl.when(i == 0)
    def _init():
        acc_ref[...] = jnp.zeros_like(acc_ref)
        statp_ref[...] = jnp.zeros_like(statp_ref)
        statl_ref[...] = jnp.zeros_like(statl_ref)
        pool_ref[...] = jnp.zeros_like(pool_ref)

    def _seg_mean():
        cnt = statp_ref[:, 3:4] + statl_ref[:, 3:4]
        return acc_ref[...] / jnp.maximum(cnt, 1.0)

    def _freeze_bias(m, l):
        bias = _dot1(m.astype(jnp.bfloat16), bmat_ref[l]) + bmsg_ref[l]
        hi, lo = _split(bias)
        bias_h_ref[...] = hi
        bias_l_ref[...] = lo
        acc_ref[...] = jnp.zeros_like(acc_ref)

    @pl.when(i == NT)
    def _b0():
        _freeze_bias(_seg_mean(), 0)

    @pl.when(i == 2 * NT)
    def _b1():
        m = _seg_mean()
        pool_ref[...] += _dot2r(m, wx_ref[0])
        _freeze_bias(m, 1)

    @pl.when(i == 3 * NT)
    def _b2():
        m = _seg_mean()
        pool_ref[...] += _dot2r(m, wx_ref[1])
        _freeze_bias(m, 2)

    @pl.when(phase == 0)
    def _embed():
        ids_c = bc_ref[0]
        oh = (jax.lax.broadcasted_iota(jnp.int32, (TILE, B), 1) == ids_c
              ).astype(jnp.bfloat16)
        oh_ref[pl.ds(row_off, TILE), :] = oh
        h0_p = _dot1(pv_ref[...].astype(jnp.bfloat16), wp_ref[...]) \
            + bp_ref[...]
        ohc = (jax.lax.broadcasted_iota(jnp.int32, (TILE, NUM_CLASSES), 1)
               == lv_ref[0]).astype(jnp.bfloat16)
        h0_l = _dot1(ohc, wl_ref[...]) + bl_ref[...]
        h0 = jnp.where(is_prot, h0_p, h0_l)
        h_ref[pl.ds(row_off, TILE), :] = h0
        acc_ref[...] += _dotT2s(oh, h0)
        po = jnp.concatenate(
            [pos_ref[...], jnp.ones((TILE, 1), jnp.float32)], axis=1)
        st = _dotT2s(oh, po)
        statp_ref[...] += jnp.where(is_prot, st, 0.0)
        statl_ref[...] += jnp.where(is_prot, 0.0, st)

    def _layer(l):
        h_old = h_ref[pl.ds(row_off, TILE), :]
        oh = oh_ref[pl.ds(row_off, TILE), :]
        lhs = jnp.concatenate([h_old.astype(jnp.bfloat16), oh, oh], axis=1)
        rhs = jnp.concatenate(
            [a_ref[l], bias_h_ref[...], bias_l_ref[...]], axis=0)
        pre = _dot1(lhs, rhs)
        h_new = h_old + pre * _sigmoid(pre)
        if l < 2:
            h_ref[pl.ds(row_off, TILE), :] = h_new
        h_hi, h_lo = _split(h_new)
        seg2 = _dotT1(oh, jnp.concatenate([h_hi, h_lo], axis=1))
        acc_ref[...] += seg2[:, :HIDDEN] + seg2[:, HIDDEN:]

    @pl.when(phase == 1)
    def _l0():
        _layer(0)

    @pl.when(phase == 2)
    def _l1():
        _layer(1)

    @pl.when(phase == 3)
    def _l2():
        _layer(2)

    @pl.when(i == 4 * NT)
    def _head():
        m3 = _seg_mean()
        pool = pool_ref[...] + _dot2r(m3, wx_ref[2])
        sum_p = statp_ref[:, 0:3]
        sum_l = statl_ref[:, 0:3]
        cnt_p = statp_ref[:, 3:4]
        cnt_all = cnt_p + statl_ref[:, 3:4]
        offset = sum_p / jnp.maximum(cnt_p, 1.0)
        pos_mean = (sum_p + sum_l - cnt_all * offset) / jnp.maximum(
            cnt_all, 1.0)
        pos_final = pos_mean + pool[:, 0:3]
        bf = jnp.bfloat16
        z1 = _dot1(pos_final.astype(bf), wh1p_ref[...]) \
            + _dot1(m3.astype(bf), wh1h_ref[...]) + bh1_ref[...]
        a1 = _ssp(z1)
        z2 = _dot1(a1.astype(bf), wh2_ref[...]) + bh2_ref[...]
        a2 = _ssp(z2)
        out_ref[...] = _dot1(a2.astype(bf), wh3_ref[...]) + bh3_ref[...]


@jax.jit
def kernel(protein_pos, protein_v, batch_protein, ligand_pos, ligand_v,
           batch_ligand, W_prot, b_prot, W_lig, b_lig, W_msg, b_msg, W_x,
           W_h1, b_h1, W_h2, b_h2, W_h3, b_h3):
    f32 = jnp.float32
    pos_all = jnp.concatenate([protein_pos, ligand_pos], axis=0)
    batch_all = jnp.concatenate([batch_protein, batch_ligand], axis=0)
    batch_c = batch_all.astype(jnp.int32).reshape(NT, TILE, 1)
    lv_c = ligand_v.astype(jnp.int32).reshape(NTL, TILE, 1)
    bf = jnp.bfloat16
    wp = jnp.pad(W_prot, ((0, 0), (0, 1))).astype(bf)
    bp = jnp.concatenate([b_prot, jnp.zeros((1,), f32)]).reshape(1, HIDDEN)
    wl = jnp.pad(W_lig, ((0, 0), (0, 1))).astype(bf)
    bl = jnp.concatenate([b_lig, jnp.ones((1,), f32)]).reshape(1, HIDDEN)
    a_w = W_msg[:, :HIDDEN, :].astype(bf)
    b_w = W_msg[:, HIDDEN:, :].astype(bf)
    bmsg = b_msg.reshape(3, 1, HIDDEN)
    wx = jnp.pad(W_x, ((0, 0), (0, 0), (0, 1))).astype(bf)
    wh1p = W_h1[0:3].astype(bf)
    wh1h = W_h1[3:].astype(bf)
    bh1 = b_h1.reshape(1, -1)
    bh2 = b_h2.reshape(1, -1)
    bh3 = b_h3.reshape(1, 1)
    wh2 = W_h2.astype(bf)
    wh3 = W_h3.astype(bf)

    def im_const(i):
        return (0, 0)

    def im_pos(i):
        return (jnp.where(i // NT == 0, jnp.minimum(i % NT, NT - 1), NT - 1),
                0)

    def im_pv(i):
        return (jnp.where(i // NT == 0, jnp.minimum(i % NT, NTP - 1),
                          NTP - 1), 0)

    def im_lv(i):
        return (jnp.where(i // NT == 0,
                          jnp.clip(i % NT - NTP, 0, NTL - 1), NTL - 1), 0, 0)

    def im_b3(i):
        return (jnp.where(i // NT == 0, jnp.minimum(i % NT, NT - 1),
                          NT - 1), 0, 0)

    in_specs = [
            pl.BlockSpec((TILE, 3), im_pos),
            pl.BlockSpec((TILE, 27), im_pv),
            pl.BlockSpec((1, TILE, 1), im_lv),
            pl.BlockSpec((1, TILE, 1), im_b3),
            pl.BlockSpec((27, HIDDEN), im_const),
            pl.BlockSpec((1, HIDDEN), im_const),
            pl.BlockSpec((NUM_CLASSES, HIDDEN), im_const),
            pl.BlockSpec((1, HIDDEN), im_const),
            pl.BlockSpec((3, HIDDEN, HIDDEN), lambda i: (0, 0, 0)),
            pl.BlockSpec((3, HIDDEN, HIDDEN), lambda i: (0, 0, 0)),
            pl.BlockSpec((3, 1, HIDDEN), lambda i: (0, 0, 0)),
            pl.BlockSpec((3, HIDDEN, 4), lambda i: (0, 0, 0)),
            pl.BlockSpec((3, 2 * HIDDEN), im_const),
            pl.BlockSpec((HIDDEN, 2 * HIDDEN), im_const),
            pl.BlockSpec((1, 2 * HIDDEN), im_const),
            pl.BlockSpec((2 * HIDDEN, HIDDEN), im_const),
            pl.BlockSpec((1, HIDDEN), im_const),
            pl.BlockSpec((HIDDEN, 1), im_const),
            pl.BlockSpec((1, 1), im_const),
        ]

    return pl.pallas_call(
        _body,
        grid=(NSTEPS,),
        in_specs=in_specs,
        out_specs=pl.BlockSpec((B, 1), im_const),
        out_shape=jax.ShapeDtypeStruct((B, 1), f32),
        scratch_shapes=[
            pltpu.VMEM((N, HIDDEN), f32),
            pltpu.VMEM((N, HIDDEN), jnp.bfloat16),
            pltpu.VMEM((B, HIDDEN), f32),
            pltpu.VMEM((B, HIDDEN), jnp.bfloat16),
            pltpu.VMEM((B, HIDDEN), jnp.bfloat16),
            pltpu.VMEM((B, 4), f32),
            pltpu.VMEM((B, 4), f32),
            pltpu.VMEM((B, 4), f32),
        ],
    )(pos_all, protein_v, lv_c, batch_c, wp, bp, wl, bl,
      a_w, b_w, bmsg, wx, wh1p, wh1h, bh1, wh2, bh2, wh3, bh3)


# speedup vs baseline: 5.5579x; 1.0031x over previous
import functools

import jax
import jax.numpy as jnp
from jax.experimental import pallas as pl
from jax.experimental.pallas import tpu as pltpu

B = 128
NUM_CLASSES = 13
HIDDEN = 128
NP = 50000
NL = 10000
N = NP + NL
TILE = 2000
NT = N // TILE
NTP = NP // TILE
NTL = NL // TILE
NSTEPS = 4 * NT + 1
_LN2 = 0.6931471805599453


def _sigmoid(x):
    return 1.0 / (1.0 + jnp.exp(-x))


def _ssp(x):
    return jnp.log1p(jnp.exp(-jnp.abs(x))) + jnp.maximum(x, 0.0) - _LN2


def _dot(a, b):
    return jax.lax.dot_general(
        a, b, (((1,), (0,)), ((), ())),
        preferred_element_type=jnp.float32,
        precision=jax.lax.Precision.HIGHEST)


def _dot1(a, b):
    return jax.lax.dot_general(
        a, b, (((1,), (0,)), ((), ())),
        preferred_element_type=jnp.float32,
        precision=jax.lax.Precision.DEFAULT)


def _split(x):
    hi = x.astype(jnp.bfloat16)
    lo = (x - hi.astype(jnp.float32)).astype(jnp.bfloat16)
    return hi, lo


def _dot2r(a, b_exact):
    a_hi, a_lo = _split(a)
    return _dot1(a_hi, b_exact) + _dot1(a_lo, b_exact)


def _dot2(a_exact, b_hi, b_lo):
    return _dot1(a_exact, b_hi) + _dot1(a_exact, b_lo)


def _dot2s(a_exact, b):
    b_hi, b_lo = _split(b)
    return _dot1(a_exact, b_hi) + _dot1(a_exact, b_lo)


def _dotT1(a, b):
    return jax.lax.dot_general(
        a, b, (((0,), (0,)), ((), ())),
        preferred_element_type=jnp.float32,
        precision=jax.lax.Precision.DEFAULT)


def _dotT2s(a_exact, b):
    b_hi, b_lo = _split(b)
    return _dotT1(a_exact, b_hi) + _dotT1(a_exact, b_lo)


def _body(pos_ref, pv_ref, lv_ref, bc_ref,
          wp_ref, bp_ref, wl_ref, bl_ref,
          a_ref, bmat_ref, bmsg_ref, wx_ref,
          wh1p_ref, wh1h_ref, bh1_ref, wh2_ref, bh2_ref, wh3_ref, bh3_ref,
          out_ref,
          h_ref, oh_ref, acc_ref, accl_ref, bias_h_ref, bias_l_ref,
          statp_ref, statl_ref, pool_ref):
    i = pl.program_id(0)
    tile = i % NT
    phase = i // NT
    row_off = pl.multiple_of(tile * TILE, TILE)
    is_prot = tile < NTP

    @pl.when(i == 0)
    def _init():
        acc_ref[...] = jnp.zeros_like(acc_ref)
        accl_ref[...] = jnp.zeros_like(accl_ref)
        statp_ref[...] = jnp.zeros_like(statp_ref)
        statl_ref[...] = jnp.zeros_like(statl_ref)
        pool_ref[...] = jnp.zeros_like(pool_ref)

    def _cnt():
        return jnp.maximum(statp_ref[:, 3:4] + statl_ref[:, 3:4], 1.0)

    def _seg_mean():
        return (acc_ref[...] + accl_ref[...]) / _cnt()

    def _seg_mean_trunc():
        return acc_ref[...] / _cnt()

    def _freeze_bias(m, l):
        bias = _dot1(m.astype(jnp.bfloat16), bmat_ref[l]) + bmsg_ref[l]
        hi, lo = _split(bias)
        bias_h_ref[...] = hi
        bias_l_ref[...] = lo
        acc_ref[...] = jnp.zeros_like(acc_ref)
        accl_ref[...] = jnp.zeros_like(accl_ref)

    @pl.when(i == NT)
    def _b0():
        _freeze_bias(_seg_mean(), 0)

    @pl.when(i == 2 * NT)
    def _b1():
        pool_ref[...] += _dot2r(_seg_mean_trunc(), wx_ref[0])
        _freeze_bias(_seg_mean(), 1)

    @pl.when(i == 3 * NT)
    def _b2():
        pool_ref[...] += _dot2r(_seg_mean_trunc(), wx_ref[1])
        _freeze_bias(_seg_mean(), 2)

    @pl.when(phase == 0)
    def _embed():
        ids_c = bc_ref[0]
        oh = (jax.lax.broadcasted_iota(jnp.int32, (TILE, B), 1) == ids_c
              ).astype(jnp.bfloat16)
        oh_ref[pl.ds(row_off, TILE), :] = oh
        ohc = (jax.lax.broadcasted_iota(jnp.int32, (TILE, NUM_CLASSES), 1)
               == lv_ref[0]).astype(jnp.bfloat16)
        zero_bf = jnp.zeros((), jnp.bfloat16)
        emb_in = jnp.concatenate(
            [jnp.where(is_prot, pv_ref[...].astype(jnp.bfloat16), zero_bf),
             jnp.where(is_prot, zero_bf, ohc)], axis=1)
        wcat = jnp.concatenate([wp_ref[...], wl_ref[...]], axis=0)
        h0 = _dot1(emb_in, wcat) \
            + jnp.where(is_prot, bp_ref[...], bl_ref[...])
        h_ref[pl.ds(row_off, TILE), :] = h0
        po = jnp.concatenate(
            [pos_ref[...], jnp.ones((TILE, 1), jnp.float32)], axis=1)
        h0_hi, h0_lo = _split(h0)
        po_hi, po_lo = _split(po)
        seg2 = _dotT1(oh, jnp.concatenate([h0_hi, h0_lo], axis=1))
        acc_ref[...] += seg2[:, :HIDDEN]
        accl_ref[...] += seg2[:, HIDDEN:]
        st4 = _dotT1(oh, jnp.concatenate([po_hi, po_lo], axis=1))
        st = st4[:, :4] + st4[:, 4:]
        statp_ref[...] += jnp.where(is_prot, st, 0.0)
        statl_ref[...] += jnp.where(is_prot, 0.0, st)

    def _layer(l):
        h_old = h_ref[pl.ds(row_off, TILE), :]
        oh = oh_ref[pl.ds(row_off, TILE), :]
        lhs = jnp.concatenate([h_old.astype(jnp.bfloat16), oh, oh], axis=1)
        rhs = jnp.concatenate(
            [a_ref[l], bias_h_ref[...], bias_l_ref[...]], axis=0)
        pre = _dot1(lhs, rhs)
        h_new = h_old + pre * _sigmoid(pre)
        if l < 2:
            h_ref[pl.ds(row_off, TILE), :] = h_new
        h_hi, h_lo = _split(h_new)
        seg2 = _dotT1(oh, jnp.concatenate([h_hi, h_lo], axis=1))
        acc_ref[...] += seg2[:, :HIDDEN]
        accl_ref[...] += seg2[:, HIDDEN:]

    @pl.when(phase == 1)
    def _l0():
        _layer(0)

    @pl.when(phase == 2)
    def _l1():
        _layer(1)

    @pl.when(phase == 3)
    def _l2():
        _layer(2)

    @pl.when(i == 4 * NT)
    def _head():
        m3 = _seg_mean()
        pool = pool_ref[...] + _dot2r(_seg_mean_trunc(), wx_ref[2])
        sum_p = statp_ref[:, 0:3]
        sum_l = statl_ref[:, 0:3]
        cnt_p = statp_ref[:, 3:4]
        cnt_all = cnt_p + statl_ref[:, 3:4]
        offset = sum_p / jnp.maximum(cnt_p, 1.0)
        pos_mean = (sum_p + sum_l - cnt_all * offset) / jnp.maximum(
            cnt_all, 1.0)
        pos_final = pos_mean + pool[:, 0:3]
        bf = jnp.bfloat16
        z1 = _dot1(pos_final.astype(bf), wh1p_ref[...]) \
            + _dot1(m3.astype(bf), wh1h_ref[...]) + bh1_ref[...]
        a1 = _ssp(z1)
        z2 = _dot1(a1.astype(bf), wh2_ref[...]) + bh2_ref[...]
        a2 = _ssp(z2)
        out_ref[...] = _dot1(a2.astype(bf), wh3_ref[...]) + bh3_ref[...]


@jax.jit
def kernel(protein_pos, protein_v, batch_protein, ligand_pos, ligand_v,
           batch_ligand, W_prot, b_prot, W_lig, b_lig, W_msg, b_msg, W_x,
           W_h1, b_h1, W_h2, b_h2, W_h3, b_h3):
    f32 = jnp.float32
    pos_all = jnp.concatenate([protein_pos, ligand_pos], axis=0)
    batch_all = jnp.concatenate([batch_protein, batch_ligand], axis=0)
    batch_c = batch_all.astype(jnp.int32).reshape(NT, TILE, 1)
    lv_c = ligand_v.astype(jnp.int32).reshape(NTL, TILE, 1)
    bf = jnp.bfloat16
    wp = jnp.pad(W_prot, ((0, 0), (0, 1))).astype(bf)
    bp = jnp.concatenate([b_prot, jnp.zeros((1,), f32)]).reshape(1, HIDDEN)
    wl = jnp.pad(W_lig, ((0, 0), (0, 1))).astype(bf)
    bl = jnp.concatenate([b_lig, jnp.ones((1,), f32)]).reshape(1, HIDDEN)
    a_w = W_msg[:, :HIDDEN, :].astype(bf)
    b_w = W_msg[:, HIDDEN:, :].astype(bf)
    bmsg = b_msg.reshape(3, 1, HIDDEN)
    wx = jnp.pad(W_x, ((0, 0), (0, 0), (0, 1))).astype(bf)
    wh1p = W_h1[0:3].astype(bf)
    wh1h = W_h1[3:].astype(bf)
    bh1 = b_h1.reshape(1, -1)
    bh2 = b_h2.reshape(1, -1)
    bh3 = b_h3.reshape(1, 1)
    wh2 = W_h2.astype(bf)
    wh3 = W_h3.astype(bf)

    def im_const(i):
        return (0, 0)

    def im_pos(i):
        return (jnp.where(i // NT == 0, jnp.minimum(i % NT, NT - 1), NT - 1),
                0)

    def im_pv(i):
        return (jnp.where(i // NT == 0, jnp.minimum(i % NT, NTP - 1),
                          NTP - 1), 0)

    def im_lv(i):
        return (jnp.where(i // NT == 0,
                          jnp.clip(i % NT - NTP, 0, NTL - 1), NTL - 1), 0, 0)

    def im_b3(i):
        return (jnp.where(i // NT == 0, jnp.minimum(i % NT, NT - 1),
                          NT - 1), 0, 0)

    in_specs = [
            pl.BlockSpec((TILE, 3), im_pos),
            pl.BlockSpec((TILE, 27), im_pv),
            pl.BlockSpec((1, TILE, 1), im_lv),
            pl.BlockSpec((1, TILE, 1), im_b3),
            pl.BlockSpec((27, HIDDEN), im_const),
            pl.BlockSpec((1, HIDDEN), im_const),
            pl.BlockSpec((NUM_CLASSES, HIDDEN), im_const),
            pl.BlockSpec((1, HIDDEN), im_const),
            pl.BlockSpec((3, HIDDEN, HIDDEN), lambda i: (0, 0, 0)),
            pl.BlockSpec((3, HIDDEN, HIDDEN), lambda i: (0, 0, 0)),
            pl.BlockSpec((3, 1, HIDDEN), lambda i: (0, 0, 0)),
            pl.BlockSpec((3, HIDDEN, 4), lambda i: (0, 0, 0)),
            pl.BlockSpec((3, 2 * HIDDEN), im_const),
            pl.BlockSpec((HIDDEN, 2 * HIDDEN), im_const),
            pl.BlockSpec((1, 2 * HIDDEN), im_const),
            pl.BlockSpec((2 * HIDDEN, HIDDEN), im_const),
            pl.BlockSpec((1, HIDDEN), im_const),
            pl.BlockSpec((HIDDEN, 1), im_const),
            pl.BlockSpec((1, 1), im_const),
        ]

    return pl.pallas_call(
        _body,
        grid=(NSTEPS,),
        in_specs=in_specs,
        out_specs=pl.BlockSpec((B, 1), im_const),
        out_shape=jax.ShapeDtypeStruct((B, 1), f32),
        scratch_shapes=[
            pltpu.VMEM((N, HIDDEN), f32),
            pltpu.VMEM((N, HIDDEN), jnp.bfloat16),
            pltpu.VMEM((B, HIDDEN), f32),
            pltpu.VMEM((B, HIDDEN), f32),
            pltpu.VMEM((B, HIDDEN), jnp.bfloat16),
            pltpu.VMEM((B, HIDDEN), jnp.bfloat16),
            pltpu.VMEM((B, 4), f32),
            pltpu.VMEM((B, 4), f32),
            pltpu.VMEM((B, 4), f32),
        ],
    )(pos_all, protein_v, lv_c, batch_c, wp, bp, wl, bl,
      a_w, b_w, bmsg, wx, wh1p, wh1h, bh1, wh2, bh2, wh3, bh3)
